# Initial kernel scaffold; baseline (speedup 1.0000x reference)
import jax
import jax.numpy as jnp
from jax.experimental import pallas as pl


def kernel(x, edge_index, W1, W2):
    raise NotImplementedError("write your pallas kernel here")


# speedup vs baseline: 14.0290x; 14.0290x over previous
import functools
import math

import jax
import jax.numpy as jnp
from jax import lax
from jax.experimental import pallas as pl
from jax.experimental.pallas import tpu as pltpu
from jax.experimental.pallas import tpu_sc as plsc

ALPHA = 0.5
THETA = 1.0
N_LAYERS = 8

NC, NS = 2, 16
NW = NC * NS
CHUNK = 128

_sc_mesh = plsc.VectorSubcoreMesh(core_axis_name="c", subcore_axis_name="s")


def _make_deg_kernel(n_acc, n_chunks):
    rows_per_tile = n_acc // NS

    @functools.partial(
        pl.kernel,
        out_type=jax.ShapeDtypeStruct((NC, n_acc, 16), jnp.float32),
        mesh=_sc_mesh,
        scratch_types=[
            pltpu.VMEM((n_chunks, CHUNK), jnp.int32),
            pltpu.VMEM((CHUNK, 16), jnp.float32),
            pltpu.VMEM_SHARED((n_acc, 16), jnp.float32),
        ],
    )
    def deg_kernel(col_hbm, ones_hbm, zeros_hbm, out_hbm, col_v, ones_v, acc_sh):
        c = lax.axis_index("c")
        s = lax.axis_index("s")
        wid = c * NS + s
        pltpu.sync_copy(col_hbm.at[wid], col_v)
        pltpu.sync_copy(ones_hbm, ones_v)
        zbase = s * rows_per_tile
        pltpu.sync_copy(zeros_hbm.at[pl.ds(zbase, rows_per_tile)],
                        acc_sh.at[pl.ds(zbase, rows_per_tile)])
        plsc.subcore_barrier()

        def body(j, carry):
            pltpu.sync_copy(ones_v, acc_sh.at[col_v.at[j]], add=True)
            return carry

        lax.fori_loop(0, n_chunks, body, 0)
        plsc.subcore_barrier()
        pltpu.sync_copy(acc_sh.at[pl.ds(zbase, rows_per_tile)],
                        out_hbm.at[c, pl.ds(zbase, rows_per_tile)])

    return deg_kernel


def _make_propagate_kernel(n, d, n_acc, n_chunks):
    rows_per_tile = n_acc // NS

    @functools.partial(
        pl.kernel,
        out_type=jax.ShapeDtypeStruct((NC, n_acc, d), jnp.float32),
        mesh=_sc_mesh,
        scratch_types=[
            pltpu.VMEM((n_chunks, CHUNK), jnp.int32),
            pltpu.VMEM((n_chunks, CHUNK), jnp.int32),
            pltpu.VMEM((CHUNK, d), jnp.float32),
            pltpu.VMEM_SHARED((n_acc, d), jnp.float32),
            pltpu.SemaphoreType.DMA,
        ],
    )
    def prop_kernel(g_hbm, row_hbm, col_hbm, zeros_hbm, out_hbm,
                    row_v, col_v, msg_v, acc_sh, sem):
        c = lax.axis_index("c")
        s = lax.axis_index("s")
        wid = c * NS + s
        pltpu.sync_copy(row_hbm.at[wid], row_v)
        pltpu.sync_copy(col_hbm.at[wid], col_v)
        zbase = s * rows_per_tile
        pltpu.sync_copy(zeros_hbm.at[pl.ds(zbase, rows_per_tile)],
                        acc_sh.at[pl.ds(zbase, rows_per_tile)])
        plsc.subcore_barrier()

        def body(j, carry):
            pltpu.async_copy(g_hbm.at[row_v.at[j]], msg_v, sem).wait()
            pltpu.sync_copy(msg_v, acc_sh.at[col_v.at[j]], add=True)
            return carry

        lax.fori_loop(0, n_chunks, body, 0)
        plsc.subcore_barrier()
        pltpu.sync_copy(acc_sh.at[pl.ds(zbase, rows_per_tile)],
                        out_hbm.at[c, pl.ds(zbase, rows_per_tile)])

    return prop_kernel


def _prologue_body(x_ref, degp_ref, dinvb_ref, g_ref, x0_ref):
    deg = degp_ref[0, :, 0:1] + degp_ref[1, :, 0:1] + 1.0
    dinv = lax.rsqrt(deg)
    dinvb = jnp.broadcast_to(dinv, x_ref.shape)
    x = x_ref[...]
    dinvb_ref[...] = dinvb
    g_ref[...] = dinvb * x
    x0_ref[...] = ALPHA * x


def _layer_body(beta, h_ref, accp_ref, dinvb_ref, x0_ref, w1_ref, w2_ref,
                h_out_ref, g_out_ref):
    dinvb = dinvb_ref[...]
    h = h_ref[...]
    acc = accp_ref[0] + accp_ref[1]
    hp = (1.0 - ALPHA) * dinvb * (acc + dinvb * h)
    x0 = x0_ref[...]
    out = (1.0 - beta) * hp
    out = out + beta * jnp.dot(hp, w1_ref[...], precision=lax.Precision.HIGHEST)
    out = out + (1.0 - beta) * x0
    out = out + beta * jnp.dot(x0, w2_ref[...], precision=lax.Precision.HIGHEST)
    h_next = jnp.maximum(out, 0.0)
    h_out_ref[...] = h_next
    g_out_ref[...] = dinvb * h_next


def kernel(x, edge_index, W1, W2):
    n, d = x.shape
    e = edge_index.shape[1]
    n_acc = -(-(n + 8) // (NS * 8)) * (NS * 8)
    pad_rows = n_acc - n
    assert d == 128

    n_chunks = -(-e // (NW * CHUNK))
    e_pad = NW * n_chunks * CHUNK
    p = e_pad - e
    pad_r = (jnp.arange(p, dtype=jnp.int32) * 37) % n
    pad_c = n + (jnp.arange(p, dtype=jnp.int32) % pad_rows)
    rowp = jnp.concatenate([edge_index[0], pad_r]).reshape(NW, n_chunks, CHUNK)
    colp = jnp.concatenate([edge_index[1], pad_c]).reshape(NW, n_chunks, CHUNK)

    ones16 = jnp.ones((CHUNK, 16), jnp.float32)
    zeros16 = jnp.zeros((n_acc, 16), jnp.float32)
    zeros128 = jnp.zeros((n_acc, d), jnp.float32)

    deg_call = _make_deg_kernel(n_acc, n_chunks)
    degp = deg_call(colp, ones16, zeros16)

    bn = 1000
    grid = (n // bn,)
    sds = jax.ShapeDtypeStruct((n, d), jnp.float32)
    dinvb, g, x0 = pl.pallas_call(
        _prologue_body,
        grid=grid,
        in_specs=[
            pl.BlockSpec((bn, d), lambda i: (i, 0)),
            pl.BlockSpec((NC, bn, 16), lambda i: (0, i, 0)),
        ],
        out_specs=[pl.BlockSpec((bn, d), lambda i: (i, 0))] * 3,
        out_shape=[sds] * 3,
    )(x, degp)

    prop_call = _make_propagate_kernel(n, d, n_acc, n_chunks)
    h = x
    for i in range(N_LAYERS):
        beta = math.log(THETA / (i + 1) + 1.0)
        accp = prop_call(g, rowp, colp, zeros128)
        h, g = pl.pallas_call(
            functools.partial(_layer_body, beta),
            grid=grid,
            in_specs=[
                pl.BlockSpec((bn, d), lambda i: (i, 0)),
                pl.BlockSpec((NC, bn, d), lambda i: (0, i, 0)),
                pl.BlockSpec((bn, d), lambda i: (i, 0)),
                pl.BlockSpec((bn, d), lambda i: (i, 0)),
                pl.BlockSpec((d, d), lambda i: (0, 0)),
                pl.BlockSpec((d, d), lambda i: (0, 0)),
            ],
            out_specs=[pl.BlockSpec((bn, d), lambda i: (i, 0))] * 2,
            out_shape=[sds] * 2,
        )(h, accp, dinvb, x0, W1[i], W2[i])
    return h
